# baseline (speedup 1.0000x reference)
import functools

import jax
import jax.numpy as jnp
from jax import lax
from jax.experimental import pallas as pl
from jax.experimental.pallas import tpu as pltpu
from jax.experimental.pallas import tpu_sc as plsc

B, S, D = 2, 2048, 2048
H = 8192
E = 8
N = B * S
T = 256
NUM_ROUTED_SLOTS = 10240
NT_ROUTED = NUM_ROUTED_SLOTS // T
NT_SHARED = N // T
SLOTS = NUM_ROUTED_SLOTS + N
NT = NT_ROUTED + NT_SHARED
CT = 512
NCHUNK = N // CT
HC = 1024
NHC = H // HC

_NEG = -1.0


def _gelu(x):
    return x * 0.5 * (1.0 + lax.erf(x * (2.0 ** -0.5)))


def _router_body(x_ref, w_ref, b_ref,
                 logits_ref, idx_ref, probs_ref, p0b_ref, p1b_ref,
                 oh1_ref, oh2_ref, off_ref, totals_ref):
    c = pl.program_id(0)
    x = x_ref[...]
    logits = jnp.dot(x, w_ref[...], preferred_element_type=jnp.float32) + b_ref[...]
    logits_ref[...] = logits
    m = jnp.max(logits, axis=1, keepdims=True)
    ex = jnp.exp(logits - m)
    probs = ex / jnp.sum(ex, axis=1, keepdims=True)

    iot = lax.broadcasted_iota(jnp.int32, (CT, E), 1).astype(jnp.float32)
    p1 = jnp.max(probs, axis=1, keepdims=True)
    i1 = jnp.min(jnp.where(probs == p1, iot, float(E)), axis=1, keepdims=True)
    masked = jnp.where(iot == i1, _NEG, probs)
    p2 = jnp.max(masked, axis=1, keepdims=True)
    i2 = jnp.min(jnp.where(masked == p2, iot, float(E)), axis=1, keepdims=True)

    pad = jnp.zeros((CT, E - 2), jnp.float32)
    idx_ref[...] = jnp.concatenate([i1, i2, pad], axis=1).astype(jnp.int32)
    probs_ref[...] = jnp.concatenate([p1, p2, pad], axis=1)
    p0b_ref[...] = jnp.broadcast_to(p1, (CT, 16))
    p1b_ref[...] = jnp.broadcast_to(p2, (CT, 16))

    oh1 = (iot == i1).astype(jnp.float32)
    oh2 = (iot == i2).astype(jnp.float32)
    oh1_ref[...] = oh1
    oh2_ref[...] = oh2
    counts = oh1 + oh2

    @pl.when(c == 0)
    def _():
        totals_ref[...] = jnp.zeros_like(totals_ref)

    rr = lax.broadcasted_iota(jnp.int32, (CT, CT), 0)
    cc = lax.broadcasted_iota(jnp.int32, (CT, CT), 1)
    tri = (cc < rr).astype(jnp.float32)
    off_local = jnp.dot(tri, counts, preferred_element_type=jnp.float32)
    off_ref[...] = off_local + totals_ref[...]
    totals_ref[...] = totals_ref[...] + jnp.sum(counts, axis=0, keepdims=True)


def _router_call(flat, router_W, router_b2d):
    return pl.pallas_call(
        _router_body,
        grid=(NCHUNK,),
        in_specs=[
            pl.BlockSpec((CT, D), lambda c: (c, 0)),
            pl.BlockSpec((D, E), lambda c: (0, 0)),
            pl.BlockSpec((1, E), lambda c: (0, 0)),
        ],
        out_specs=[
            pl.BlockSpec((CT, E), lambda c: (c, 0)),
            pl.BlockSpec((CT, E), lambda c: (c, 0)),
            pl.BlockSpec((CT, E), lambda c: (c, 0)),
            pl.BlockSpec((CT, 16), lambda c: (c, 0)),
            pl.BlockSpec((CT, 16), lambda c: (c, 0)),
            pl.BlockSpec((CT, E), lambda c: (c, 0)),
            pl.BlockSpec((CT, E), lambda c: (c, 0)),
            pl.BlockSpec((CT, E), lambda c: (c, 0)),
            pl.BlockSpec((1, E), lambda c: (0, 0)),
        ],
        out_shape=[
            jax.ShapeDtypeStruct((N, E), jnp.float32),
            jax.ShapeDtypeStruct((N, E), jnp.int32),
            jax.ShapeDtypeStruct((N, E), jnp.float32),
            jax.ShapeDtypeStruct((N, 16), jnp.float32),
            jax.ShapeDtypeStruct((N, 16), jnp.float32),
            jax.ShapeDtypeStruct((N, E), jnp.float32),
            jax.ShapeDtypeStruct((N, E), jnp.float32),
            jax.ShapeDtypeStruct((N, E), jnp.float32),
            jax.ShapeDtypeStruct((1, E), jnp.float32),
        ],
    )(flat, router_W, router_b2d)


def _dst_body(oh1_ref, oh2_ref, off_ref, totals_ref,
              dst0_ref, dst1_ref, te_ref):
    tot = totals_ref[...]
    padded = jnp.ceil(tot * (1.0 / T)) * float(T)
    ru = lax.broadcasted_iota(jnp.int32, (E, E), 0)
    cu = lax.broadcasted_iota(jnp.int32, (E, E), 1)
    triu = (ru < cu).astype(jnp.float32)
    base = jnp.dot(padded, triu, preferred_element_type=jnp.float32)

    off = off_ref[...] + base
    d0 = jnp.sum(oh1_ref[...] * off, axis=1)
    d1 = jnp.sum(oh2_ref[...] * off, axis=1)
    dst0_ref[...] = d0.reshape(32, 128).astype(jnp.int32)
    dst1_ref[...] = d1.reshape(32, 128).astype(jnp.int32)

    eye = (ru == cu).astype(jnp.float32)
    base_col = jnp.sum(jnp.dot(jnp.ones((E, 1), jnp.float32), base,
                               preferred_element_type=jnp.float32) * eye,
                       axis=1, keepdims=True)
    jT = lax.broadcasted_iota(jnp.int32, (E, 64), 1).astype(jnp.float32) * float(T)
    te = jnp.sum((jT >= base_col).astype(jnp.float32), axis=0, keepdims=True) - 1.0
    total_padded = jnp.sum(padded, axis=1, keepdims=True)
    jT1 = lax.broadcasted_iota(jnp.int32, (1, 64), 1).astype(jnp.float32) * float(T)
    te = jnp.where(jT1 >= total_padded, float(E), te)
    te_ref[...] = te.astype(jnp.int32)


def _dst_call(oh1, oh2, off, totals):
    return pl.pallas_call(
        _dst_body,
        in_specs=[
            pl.BlockSpec((N, E), lambda: (0, 0)),
            pl.BlockSpec((N, E), lambda: (0, 0)),
            pl.BlockSpec((N, E), lambda: (0, 0)),
            pl.BlockSpec((1, E), lambda: (0, 0)),
        ],
        out_specs=[
            pl.BlockSpec((32, 128), lambda: (0, 0)),
            pl.BlockSpec((32, 128), lambda: (0, 0)),
            pl.BlockSpec((1, 64), lambda: (0, 0)),
        ],
        out_shape=[
            jax.ShapeDtypeStruct((32, 128), jnp.int32),
            jax.ShapeDtypeStruct((32, 128), jnp.int32),
            jax.ShapeDtypeStruct((1, 64), jnp.int32),
        ],
    )(oh1, oh2, off, totals)


def _sc_mesh():
    return plsc.VectorSubcoreMesh(core_axis_name="c", subcore_axis_name="s")


_NCORES = 2
_NSUB = 16
_NW = _NCORES * _NSUB
_TPW = N // _NW
_DCH = 32
_CCH = 16


def _dispatch_body(flat_hbm, dst0_hbm, dst1_hbm, xg_hbm,
                   rows_v, idx0_v, idx1_v, sem):
    wid = lax.axis_index("s") * _NCORES + lax.axis_index("c")
    base = wid * _TPW
    for j in range(_TPW // _DCH):
        b = base + j * _DCH
        pltpu.sync_copy(flat_hbm.at[pl.ds(b, _DCH)], rows_v)
        pltpu.sync_copy(dst0_hbm.at[pl.ds(b, _DCH)], idx0_v)
        pltpu.sync_copy(dst1_hbm.at[pl.ds(b, _DCH)], idx1_v)
        c0 = pltpu.async_copy(rows_v, xg_hbm.at[idx0_v], sem)
        c1 = pltpu.async_copy(rows_v, xg_hbm.at[idx1_v], sem)
        c0.wait()
        c1.wait()


def _dispatch_call(flat_i32, dst0, dst1):
    f = functools.partial(
        pl.kernel,
        mesh=_sc_mesh(),
        out_type=jax.ShapeDtypeStruct((NUM_ROUTED_SLOTS, D // 2), jnp.int32),
        scratch_types=[
            pltpu.VMEM((_DCH, D // 2), jnp.int32),
            pltpu.VMEM((_DCH,), jnp.int32),
            pltpu.VMEM((_DCH,), jnp.int32),
            pltpu.SemaphoreType.DMA,
        ],
    )(_dispatch_body)
    return f(flat_i32, dst0, dst1)


def _ffn_body_first(te_ref, x_ref, w1_ref, b1_ref, w2_ref, b2_ref, out_ref):
    tv = te_ref[pl.program_id(0)]

    @pl.when(tv < E)
    def _():
        x = x_ref[...].astype(jnp.float32)
        mid = _gelu(jnp.dot(x, w1_ref[0], preferred_element_type=jnp.float32)
                    + b1_ref[0, 0])
        out_ref[...] = (jnp.dot(mid, w2_ref[0],
                                preferred_element_type=jnp.float32)
                        + b2_ref[0])


def _ffn_body_rest(te_ref, x_ref, w1_ref, b1_ref, w2_ref, prev_ref, out_ref):
    tv = te_ref[pl.program_id(0)]

    @pl.when(tv < E)
    def _():
        x = x_ref[...].astype(jnp.float32)
        mid = _gelu(jnp.dot(x, w1_ref[0], preferred_element_type=jnp.float32)
                    + b1_ref[0, 0])
        out_ref[...] = prev_ref[...] + jnp.dot(
            mid, w2_ref[0], preferred_element_type=jnp.float32)


def _ffn_call(hc, te_r, xg, rW1, rb1, rW2, rb2, prev):
    first = prev is None

    def _e(te, i):
        return jnp.minimum(te[i], E - 1)

    in_specs = [
        pl.BlockSpec((T, D), lambda i, te: (i, 0)),
        pl.BlockSpec((1, D, HC), lambda i, te, _hc=hc: (_e(te, i), 0, _hc)),
        pl.BlockSpec((1, 1, 1, HC),
                     lambda i, te, _hc=hc: (_e(te, i), _hc, 0, 0)),
        pl.BlockSpec((1, HC, D), lambda i, te, _hc=hc: (_e(te, i), _hc, 0)),
    ]
    args = [xg, rW1, rb1.reshape(E, NHC, 1, HC), rW2]
    if first:
        body = _ffn_body_first
        in_specs.append(pl.BlockSpec((1, 1, D), lambda i, te: (_e(te, i), 0, 0)))
        args.append(rb2.reshape(E, 1, D))
    else:
        body = _ffn_body_rest
        in_specs.append(pl.BlockSpec((T, D), lambda i, te: (i, 0)))
        args.append(prev)
    return pl.pallas_call(
        body,
        grid_spec=pltpu.PrefetchScalarGridSpec(
            num_scalar_prefetch=1,
            grid=(NT_ROUTED,),
            in_specs=in_specs,
            out_specs=pl.BlockSpec((T, D), lambda i, te: (i, 0)),
        ),
        out_shape=jax.ShapeDtypeStruct((NUM_ROUTED_SLOTS, D), jnp.float32),
    )(te_r, *args)


def _sffn_body_first(x_ref, w1_ref, b1_ref, w2_ref, b2_ref, out_ref):
    x = x_ref[...].astype(jnp.float32)
    mid = _gelu(jnp.dot(x, w1_ref[...], preferred_element_type=jnp.float32)
                + b1_ref[...])
    out_ref[...] = (jnp.dot(mid, w2_ref[...],
                            preferred_element_type=jnp.float32) + b2_ref[...])


def _sffn_body_rest(x_ref, w1_ref, b1_ref, w2_ref, prev_ref, out_ref):
    x = x_ref[...].astype(jnp.float32)
    mid = _gelu(jnp.dot(x, w1_ref[...], preferred_element_type=jnp.float32)
                + b1_ref[...])
    out_ref[...] = prev_ref[...] + jnp.dot(
        mid, w2_ref[...], preferred_element_type=jnp.float32)


def _sffn_call(hc, flat, sW1, sb1, sW2, sb2, prev):
    first = prev is None
    in_specs = [
        pl.BlockSpec((T, D), lambda i: (i, 0)),
        pl.BlockSpec((D, HC), lambda i, _hc=hc: (0, _hc)),
        pl.BlockSpec((1, HC), lambda i, _hc=hc: (0, _hc)),
        pl.BlockSpec((HC, D), lambda i, _hc=hc: (_hc, 0)),
    ]
    args = [flat, sW1[0], sb1, sW2[0]]
    if first:
        body = _sffn_body_first
        in_specs.append(pl.BlockSpec((1, D), lambda i: (0, 0)))
        args.append(sb2)
    else:
        body = _sffn_body_rest
        in_specs.append(pl.BlockSpec((T, D), lambda i: (i, 0)))
        args.append(prev)
    return pl.pallas_call(
        body,
        grid=(NT_SHARED,),
        in_specs=in_specs,
        out_specs=pl.BlockSpec((T, D), lambda i: (i, 0)),
        out_shape=jax.ShapeDtypeStruct((N, D), jnp.float32),
    )(*args)


def _combine_body(y_hbm, ys_hbm, dst0_hbm, dst1_hbm, p0b_hbm, p1b_hbm,
                  out_hbm, acc_v, buf0_v, buf1_v, idx0_v, idx1_v, w0_v, w1_v,
                  sem):
    wid = lax.axis_index("s") * _NCORES + lax.axis_index("c")
    base = wid * _TPW
    for j in range(_TPW // _CCH):
        b = base + j * _CCH
        pltpu.sync_copy(dst0_hbm.at[pl.ds(b, _CCH)], idx0_v)
        pltpu.sync_copy(dst1_hbm.at[pl.ds(b, _CCH)], idx1_v)
        pltpu.sync_copy(p0b_hbm.at[pl.ds(b, _CCH)], w0_v)
        pltpu.sync_copy(p1b_hbm.at[pl.ds(b, _CCH)], w1_v)
        pltpu.sync_copy(ys_hbm.at[pl.ds(b, _CCH)], acc_v)
        g0 = pltpu.async_copy(y_hbm.at[idx0_v], buf0_v, sem)
        g1 = pltpu.async_copy(y_hbm.at[idx1_v], buf1_v, sem)
        g0.wait()
        g1.wait()

        def row_body(r, _):
            w0 = w0_v[r]
            w1 = w1_v[r]

            def col_body(cb, __):
                for u in range(4):
                    sl = pl.ds(cb * 64 + u * 16, 16)
                    acc_v.at[r][sl] = (acc_v.at[r][sl]
                                       + w0 * buf0_v.at[r][sl]
                                       + w1 * buf1_v.at[r][sl])
                return __

            return lax.fori_loop(0, D // 64, col_body, _)

        lax.fori_loop(0, _CCH, row_body, 0)
        pltpu.sync_copy(acc_v, out_hbm.at[pl.ds(b, _CCH)])


def _combine_call(y, ys, dst0, dst1, p0b, p1b):
    f = functools.partial(
        pl.kernel,
        mesh=_sc_mesh(),
        out_type=jax.ShapeDtypeStruct((N, D), jnp.float32),
        scratch_types=[
            pltpu.VMEM((_CCH, D), jnp.float32),
            pltpu.VMEM((_CCH, D), jnp.float32),
            pltpu.VMEM((_CCH, D), jnp.float32),
            pltpu.VMEM((_CCH,), jnp.int32),
            pltpu.VMEM((_CCH,), jnp.int32),
            pltpu.VMEM((_CCH, 16), jnp.float32),
            pltpu.VMEM((_CCH, 16), jnp.float32),
            pltpu.SemaphoreType.DMA,
        ],
    )(_combine_body)
    return f(y, ys, dst0, dst1, p0b, p1b)


def kernel(hidden, router_W, router_b, shared_W1, shared_b1, shared_W2,
           shared_b2, routed_W1, routed_b1, routed_W2, routed_b2):
    flat = hidden.reshape(N, D)

    (logits, idx8, probs8, p0b, p1b,
     oh1, oh2, off, totals) = _router_call(flat, router_W,
                                           router_b.reshape(1, E))
    dst0m, dst1m, te_pad = _dst_call(oh1, oh2, off, totals)
    dst0 = dst0m.reshape(N)
    dst1 = dst1m.reshape(N)
    te_r = te_pad.reshape(64)[:NT_ROUTED]
    flat_bf = flat.astype(jnp.bfloat16)
    flat_i32 = lax.bitcast_convert_type(
        flat_bf.reshape(N, D // 2, 2), jnp.int32)

    xg_i32 = _dispatch_call(flat_i32, dst0, dst1)
    xg = lax.bitcast_convert_type(xg_i32, jnp.bfloat16).reshape(
        NUM_ROUTED_SLOTS, D)

    y = None
    ys = None
    for hc in range(NHC):
        y = _ffn_call(hc, te_r, xg, routed_W1, routed_b1, routed_W2,
                      routed_b2, y)
        ys = _sffn_call(hc, flat_bf, shared_W1, shared_b1, shared_W2,
                        shared_b2, ys)

    out_flat = _combine_call(y, ys, dst0, dst1, p0b, p1b)

    output = out_flat.reshape(B, S, D)
    aux_logits = logits.reshape(B, S, E)
    aux_idx = idx8[:, :2].reshape(B, S, 2)
    aux_probs = probs8[:, :2].reshape(B, S, 2)
    return output, aux_logits, aux_idx, aux_probs


# speedup vs baseline: 1.2896x; 1.2896x over previous
import functools

import jax
import jax.numpy as jnp
from jax import lax
from jax.experimental import pallas as pl
from jax.experimental.pallas import tpu as pltpu
from jax.experimental.pallas import tpu_sc as plsc

B, S, D = 2, 2048, 2048
H = 8192
E = 8
N = B * S
T = 256
NUM_ROUTED_SLOTS = 10240
NT_ROUTED = NUM_ROUTED_SLOTS // T
NT_SHARED = N // T
SLOTS = NUM_ROUTED_SLOTS + N
NT = NT_ROUTED + NT_SHARED
CT = 512
NCHUNK = N // CT
HC = 1024
NHC = H // HC

_NEG = -1.0


def _gelu(x):
    return x * 0.5 * (1.0 + lax.erf(x * (2.0 ** -0.5)))


def _router_body(x_ref, w_ref, b_ref,
                 logits_ref, idx_ref, probs_ref, p0b_ref, p1b_ref,
                 oh1_ref, oh2_ref, off_ref, totals_ref):
    c = pl.program_id(0)
    x = x_ref[...]
    logits = jnp.dot(x, w_ref[...], preferred_element_type=jnp.float32) + b_ref[...]
    logits_ref[...] = logits
    m = jnp.max(logits, axis=1, keepdims=True)
    ex = jnp.exp(logits - m)
    probs = ex / jnp.sum(ex, axis=1, keepdims=True)

    iot = lax.broadcasted_iota(jnp.int32, (CT, E), 1).astype(jnp.float32)
    p1 = jnp.max(probs, axis=1, keepdims=True)
    i1 = jnp.min(jnp.where(probs == p1, iot, float(E)), axis=1, keepdims=True)
    masked = jnp.where(iot == i1, _NEG, probs)
    p2 = jnp.max(masked, axis=1, keepdims=True)
    i2 = jnp.min(jnp.where(masked == p2, iot, float(E)), axis=1, keepdims=True)

    pad = jnp.zeros((CT, E - 2), jnp.float32)
    idx_ref[...] = jnp.concatenate([i1, i2, pad], axis=1).astype(jnp.int32)
    probs_ref[...] = jnp.concatenate([p1, p2, pad], axis=1)
    p0b_ref[...] = jnp.broadcast_to(p1, (CT, 16))
    p1b_ref[...] = jnp.broadcast_to(p2, (CT, 16))

    oh1 = (iot == i1).astype(jnp.float32)
    oh2 = (iot == i2).astype(jnp.float32)
    oh1_ref[...] = oh1
    oh2_ref[...] = oh2
    counts = oh1 + oh2

    @pl.when(c == 0)
    def _():
        totals_ref[...] = jnp.zeros_like(totals_ref)

    rr = lax.broadcasted_iota(jnp.int32, (CT, CT), 0)
    cc = lax.broadcasted_iota(jnp.int32, (CT, CT), 1)
    tri = (cc < rr).astype(jnp.float32)
    off_local = jnp.dot(tri, counts, preferred_element_type=jnp.float32)
    off_ref[...] = off_local + totals_ref[...]
    totals_ref[...] = totals_ref[...] + jnp.sum(counts, axis=0, keepdims=True)


def _router_call(flat, router_W, router_b2d):
    return pl.pallas_call(
        _router_body,
        grid=(NCHUNK,),
        in_specs=[
            pl.BlockSpec((CT, D), lambda c: (c, 0)),
            pl.BlockSpec((D, E), lambda c: (0, 0)),
            pl.BlockSpec((1, E), lambda c: (0, 0)),
        ],
        out_specs=[
            pl.BlockSpec((CT, E), lambda c: (c, 0)),
            pl.BlockSpec((CT, E), lambda c: (c, 0)),
            pl.BlockSpec((CT, E), lambda c: (c, 0)),
            pl.BlockSpec((CT, 16), lambda c: (c, 0)),
            pl.BlockSpec((CT, 16), lambda c: (c, 0)),
            pl.BlockSpec((CT, E), lambda c: (c, 0)),
            pl.BlockSpec((CT, E), lambda c: (c, 0)),
            pl.BlockSpec((CT, E), lambda c: (c, 0)),
            pl.BlockSpec((1, E), lambda c: (0, 0)),
        ],
        out_shape=[
            jax.ShapeDtypeStruct((N, E), jnp.float32),
            jax.ShapeDtypeStruct((N, E), jnp.int32),
            jax.ShapeDtypeStruct((N, E), jnp.float32),
            jax.ShapeDtypeStruct((N, 16), jnp.float32),
            jax.ShapeDtypeStruct((N, 16), jnp.float32),
            jax.ShapeDtypeStruct((N, E), jnp.float32),
            jax.ShapeDtypeStruct((N, E), jnp.float32),
            jax.ShapeDtypeStruct((N, E), jnp.float32),
            jax.ShapeDtypeStruct((1, E), jnp.float32),
        ],
    )(flat, router_W, router_b2d)


def _dst_body(oh1_ref, oh2_ref, off_ref, totals_ref,
              dst0_ref, dst1_ref, te_ref):
    tot = totals_ref[...]
    padded = jnp.ceil(tot * (1.0 / T)) * float(T)
    ru = lax.broadcasted_iota(jnp.int32, (E, E), 0)
    cu = lax.broadcasted_iota(jnp.int32, (E, E), 1)
    triu = (ru < cu).astype(jnp.float32)
    base = jnp.dot(padded, triu, preferred_element_type=jnp.float32)

    off = off_ref[...] + base
    d0 = jnp.sum(oh1_ref[...] * off, axis=1)
    d1 = jnp.sum(oh2_ref[...] * off, axis=1)
    dst0_ref[...] = d0.reshape(32, 128).astype(jnp.int32)
    dst1_ref[...] = d1.reshape(32, 128).astype(jnp.int32)

    eye = (ru == cu).astype(jnp.float32)
    base_col = jnp.sum(jnp.dot(jnp.ones((E, 1), jnp.float32), base,
                               preferred_element_type=jnp.float32) * eye,
                       axis=1, keepdims=True)
    jT = lax.broadcasted_iota(jnp.int32, (E, 64), 1).astype(jnp.float32) * float(T)
    te = jnp.sum((jT >= base_col).astype(jnp.float32), axis=0, keepdims=True) - 1.0
    total_padded = jnp.sum(padded, axis=1, keepdims=True)
    jT1 = lax.broadcasted_iota(jnp.int32, (1, 64), 1).astype(jnp.float32) * float(T)
    te = jnp.where(jT1 >= total_padded, float(E), te)
    te_ref[...] = te.astype(jnp.int32)


def _dst_call(oh1, oh2, off, totals):
    return pl.pallas_call(
        _dst_body,
        in_specs=[
            pl.BlockSpec((N, E), lambda: (0, 0)),
            pl.BlockSpec((N, E), lambda: (0, 0)),
            pl.BlockSpec((N, E), lambda: (0, 0)),
            pl.BlockSpec((1, E), lambda: (0, 0)),
        ],
        out_specs=[
            pl.BlockSpec((32, 128), lambda: (0, 0)),
            pl.BlockSpec((32, 128), lambda: (0, 0)),
            pl.BlockSpec((1, 64), lambda: (0, 0)),
        ],
        out_shape=[
            jax.ShapeDtypeStruct((32, 128), jnp.int32),
            jax.ShapeDtypeStruct((32, 128), jnp.int32),
            jax.ShapeDtypeStruct((1, 64), jnp.int32),
        ],
    )(oh1, oh2, off, totals)


def _sc_mesh():
    return plsc.VectorSubcoreMesh(core_axis_name="c", subcore_axis_name="s")


_NCORES = 2
_NSUB = 16
_NW = _NCORES * _NSUB
_TPW = N // _NW
_DCH = 32
_CCH = 16


def _dispatch_body(flat_hbm, dst0_hbm, dst1_hbm, xg_hbm,
                   rows_v, idx0_v, idx1_v, sem):
    wid = lax.axis_index("s") * _NCORES + lax.axis_index("c")
    base = wid * _TPW
    for j in range(_TPW // _DCH):
        b = base + j * _DCH
        pltpu.sync_copy(flat_hbm.at[pl.ds(b, _DCH)], rows_v)
        pltpu.sync_copy(dst0_hbm.at[pl.ds(b, _DCH)], idx0_v)
        pltpu.sync_copy(dst1_hbm.at[pl.ds(b, _DCH)], idx1_v)
        c0 = pltpu.async_copy(rows_v, xg_hbm.at[idx0_v], sem)
        c1 = pltpu.async_copy(rows_v, xg_hbm.at[idx1_v], sem)
        c0.wait()
        c1.wait()


def _dispatch_call(flat, dst0, dst1):
    f = functools.partial(
        pl.kernel,
        mesh=_sc_mesh(),
        out_type=jax.ShapeDtypeStruct((NUM_ROUTED_SLOTS, D), jnp.float32),
        scratch_types=[
            pltpu.VMEM((_DCH, D), jnp.float32),
            pltpu.VMEM((_DCH,), jnp.int32),
            pltpu.VMEM((_DCH,), jnp.int32),
            pltpu.SemaphoreType.DMA,
        ],
    )(_dispatch_body)
    return f(flat, dst0, dst1)


def _ffn_body_first(te_ref, x_ref, w1_ref, b1_ref, w2_ref, b2_ref, out_ref):
    tv = te_ref[pl.program_id(0)]

    @pl.when(tv < E)
    def _():
        x = x_ref[...]
        mid = _gelu(jnp.dot(x, w1_ref[0], preferred_element_type=jnp.float32)
                    + b1_ref[0, 0])
        out_ref[...] = (jnp.dot(mid, w2_ref[0],
                                preferred_element_type=jnp.float32)
                        + b2_ref[0])


def _ffn_body_rest(te_ref, x_ref, w1_ref, b1_ref, w2_ref, prev_ref, out_ref):
    tv = te_ref[pl.program_id(0)]

    @pl.when(tv < E)
    def _():
        x = x_ref[...]
        mid = _gelu(jnp.dot(x, w1_ref[0], preferred_element_type=jnp.float32)
                    + b1_ref[0, 0])
        out_ref[...] = prev_ref[...] + jnp.dot(
            mid, w2_ref[0], preferred_element_type=jnp.float32)


def _ffn_call(hc, te_r, xg, rW1, rb1, rW2, rb2, prev):
    first = prev is None

    def _e(te, i):
        return jnp.minimum(te[i], E - 1)

    in_specs = [
        pl.BlockSpec((T, D), lambda i, te: (i, 0)),
        pl.BlockSpec((1, D, HC), lambda i, te, _hc=hc: (_e(te, i), 0, _hc)),
        pl.BlockSpec((1, 1, 1, HC),
                     lambda i, te, _hc=hc: (_e(te, i), _hc, 0, 0)),
        pl.BlockSpec((1, HC, D), lambda i, te, _hc=hc: (_e(te, i), _hc, 0)),
    ]
    args = [xg, rW1, rb1.reshape(E, NHC, 1, HC), rW2]
    if first:
        body = _ffn_body_first
        in_specs.append(pl.BlockSpec((1, 1, D), lambda i, te: (_e(te, i), 0, 0)))
        args.append(rb2.reshape(E, 1, D))
    else:
        body = _ffn_body_rest
        in_specs.append(pl.BlockSpec((T, D), lambda i, te: (i, 0)))
        args.append(prev)
    return pl.pallas_call(
        body,
        grid_spec=pltpu.PrefetchScalarGridSpec(
            num_scalar_prefetch=1,
            grid=(NT_ROUTED,),
            in_specs=in_specs,
            out_specs=pl.BlockSpec((T, D), lambda i, te: (i, 0)),
        ),
        out_shape=jax.ShapeDtypeStruct((NUM_ROUTED_SLOTS, D), jnp.float32),
    )(te_r, *args)


def _sffn_body_first(x_ref, w1_ref, b1_ref, w2_ref, b2_ref, out_ref):
    x = x_ref[...]
    mid = _gelu(jnp.dot(x, w1_ref[...], preferred_element_type=jnp.float32)
                + b1_ref[...])
    out_ref[...] = (jnp.dot(mid, w2_ref[...],
                            preferred_element_type=jnp.float32) + b2_ref[...])


def _sffn_body_rest(x_ref, w1_ref, b1_ref, w2_ref, prev_ref, out_ref):
    x = x_ref[...]
    mid = _gelu(jnp.dot(x, w1_ref[...], preferred_element_type=jnp.float32)
                + b1_ref[...])
    out_ref[...] = prev_ref[...] + jnp.dot(
        mid, w2_ref[...], preferred_element_type=jnp.float32)


def _sffn_call(hc, flat, sW1, sb1, sW2, sb2, prev):
    first = prev is None
    in_specs = [
        pl.BlockSpec((T, D), lambda i: (i, 0)),
        pl.BlockSpec((D, HC), lambda i, _hc=hc: (0, _hc)),
        pl.BlockSpec((1, HC), lambda i, _hc=hc: (0, _hc)),
        pl.BlockSpec((HC, D), lambda i, _hc=hc: (_hc, 0)),
    ]
    args = [flat, sW1[0], sb1, sW2[0]]
    if first:
        body = _sffn_body_first
        in_specs.append(pl.BlockSpec((1, D), lambda i: (0, 0)))
        args.append(sb2)
    else:
        body = _sffn_body_rest
        in_specs.append(pl.BlockSpec((T, D), lambda i: (i, 0)))
        args.append(prev)
    return pl.pallas_call(
        body,
        grid=(NT_SHARED,),
        in_specs=in_specs,
        out_specs=pl.BlockSpec((T, D), lambda i: (i, 0)),
        out_shape=jax.ShapeDtypeStruct((N, D), jnp.float32),
    )(*args)


def _combine_body(y_hbm, ys_hbm, dst0_hbm, dst1_hbm, p0b_hbm, p1b_hbm,
                  out_hbm, acc_v, buf0_v, buf1_v, idx0_v, idx1_v, w0_v, w1_v,
                  sem):
    wid = lax.axis_index("s") * _NCORES + lax.axis_index("c")
    base = wid * _TPW
    for j in range(_TPW // _CCH):
        b = base + j * _CCH
        pltpu.sync_copy(dst0_hbm.at[pl.ds(b, _CCH)], idx0_v)
        pltpu.sync_copy(dst1_hbm.at[pl.ds(b, _CCH)], idx1_v)
        pltpu.sync_copy(p0b_hbm.at[pl.ds(b, _CCH)], w0_v)
        pltpu.sync_copy(p1b_hbm.at[pl.ds(b, _CCH)], w1_v)
        pltpu.sync_copy(ys_hbm.at[pl.ds(b, _CCH)], acc_v)
        g0 = pltpu.async_copy(y_hbm.at[idx0_v], buf0_v, sem)
        g1 = pltpu.async_copy(y_hbm.at[idx1_v], buf1_v, sem)
        g0.wait()
        g1.wait()

        def row_body(r, _):
            w0 = w0_v[r]
            w1 = w1_v[r]

            def col_body(cb, __):
                for u in range(4):
                    sl = pl.ds(cb * 64 + u * 16, 16)
                    acc_v.at[r][sl] = (acc_v.at[r][sl]
                                       + w0 * buf0_v.at[r][sl]
                                       + w1 * buf1_v.at[r][sl])
                return __

            return lax.fori_loop(0, D // 64, col_body, _)

        lax.fori_loop(0, _CCH, row_body, 0)
        pltpu.sync_copy(acc_v, out_hbm.at[pl.ds(b, _CCH)])


def _combine_call(y, ys, dst0, dst1, p0b, p1b):
    f = functools.partial(
        pl.kernel,
        mesh=_sc_mesh(),
        out_type=jax.ShapeDtypeStruct((N, D), jnp.float32),
        scratch_types=[
            pltpu.VMEM((_CCH, D), jnp.float32),
            pltpu.VMEM((_CCH, D), jnp.float32),
            pltpu.VMEM((_CCH, D), jnp.float32),
            pltpu.VMEM((_CCH,), jnp.int32),
            pltpu.VMEM((_CCH,), jnp.int32),
            pltpu.VMEM((_CCH, 16), jnp.float32),
            pltpu.VMEM((_CCH, 16), jnp.float32),
            pltpu.SemaphoreType.DMA,
        ],
    )(_combine_body)
    return f(y, ys, dst0, dst1, p0b, p1b)


def kernel(hidden, router_W, router_b, shared_W1, shared_b1, shared_W2,
           shared_b2, routed_W1, routed_b1, routed_W2, routed_b2):
    flat = hidden.reshape(N, D)

    (logits, idx8, probs8, p0b, p1b,
     oh1, oh2, off, totals) = _router_call(flat, router_W,
                                           router_b.reshape(1, E))
    dst0m, dst1m, te_pad = _dst_call(oh1, oh2, off, totals)
    dst0 = dst0m.reshape(N)
    dst1 = dst1m.reshape(N)
    te_r = te_pad.reshape(64)[:NT_ROUTED]

    xg = _dispatch_call(flat, dst0, dst1)

    y = None
    ys = None
    for hc in range(NHC):
        y = _ffn_call(hc, te_r, xg, routed_W1, routed_b1, routed_W2,
                      routed_b2, y)
        ys = _sffn_call(hc, flat, shared_W1, shared_b1, shared_W2,
                        shared_b2, ys)

    out_flat = _combine_call(y, ys, dst0, dst1, p0b, p1b)

    output = out_flat.reshape(B, S, D)
    aux_logits = logits.reshape(B, S, E)
    aux_idx = idx8[:, :2].reshape(B, S, 2)
    aux_probs = probs8[:, :2].reshape(B, S, 2)
    return output, aux_logits, aux_idx, aux_probs


# speedup vs baseline: 1.3519x; 1.0483x over previous
import functools

import jax
import jax.numpy as jnp
from jax import lax
from jax.experimental import pallas as pl
from jax.experimental.pallas import tpu as pltpu
from jax.experimental.pallas import tpu_sc as plsc

B, S, D = 2, 2048, 2048
H = 8192
E = 8
N = B * S
T = 256
NUM_ROUTED_SLOTS = 10240
NT_ROUTED = NUM_ROUTED_SLOTS // T
NT_SHARED = N // T
SLOTS = NUM_ROUTED_SLOTS + N
NT = NT_ROUTED + NT_SHARED
CT = 512
NCHUNK = N // CT
HC = 1024
NHC = H // HC

_NEG = -1.0


def _gelu(x):
    return x * 0.5 * (1.0 + lax.erf(x * (2.0 ** -0.5)))


def _router_body(x_ref, w_ref, b_ref,
                 logits_ref, idx_ref, probs_ref, p0b_ref, p1b_ref,
                 oh1_ref, oh2_ref, off_ref, totals_ref):
    c = pl.program_id(0)
    x = x_ref[...]
    logits = jnp.dot(x, w_ref[...], preferred_element_type=jnp.float32) + b_ref[...]
    logits_ref[...] = logits
    m = jnp.max(logits, axis=1, keepdims=True)
    ex = jnp.exp(logits - m)
    probs = ex / jnp.sum(ex, axis=1, keepdims=True)

    iot = lax.broadcasted_iota(jnp.int32, (CT, E), 1).astype(jnp.float32)
    p1 = jnp.max(probs, axis=1, keepdims=True)
    i1 = jnp.min(jnp.where(probs == p1, iot, float(E)), axis=1, keepdims=True)
    masked = jnp.where(iot == i1, _NEG, probs)
    p2 = jnp.max(masked, axis=1, keepdims=True)
    i2 = jnp.min(jnp.where(masked == p2, iot, float(E)), axis=1, keepdims=True)

    pad = jnp.zeros((CT, E - 2), jnp.float32)
    idx_ref[...] = jnp.concatenate([i1, i2, pad], axis=1).astype(jnp.int32)
    probs_ref[...] = jnp.concatenate([p1, p2, pad], axis=1)
    p0b_ref[...] = jnp.broadcast_to(p1, (CT, 16))
    p1b_ref[...] = jnp.broadcast_to(p2, (CT, 16))

    oh1 = (iot == i1).astype(jnp.float32)
    oh2 = (iot == i2).astype(jnp.float32)
    oh1_ref[...] = oh1
    oh2_ref[...] = oh2
    counts = oh1 + oh2

    @pl.when(c == 0)
    def _():
        totals_ref[...] = jnp.zeros_like(totals_ref)

    rr = lax.broadcasted_iota(jnp.int32, (CT, CT), 0)
    cc = lax.broadcasted_iota(jnp.int32, (CT, CT), 1)
    tri = (cc < rr).astype(jnp.float32)
    off_local = jnp.dot(tri, counts, preferred_element_type=jnp.float32)
    off_ref[...] = off_local + totals_ref[...]
    totals_ref[...] = totals_ref[...] + jnp.sum(counts, axis=0, keepdims=True)


def _router_call(flat, router_W, router_b2d):
    return pl.pallas_call(
        _router_body,
        grid=(NCHUNK,),
        in_specs=[
            pl.BlockSpec((CT, D), lambda c: (c, 0)),
            pl.BlockSpec((D, E), lambda c: (0, 0)),
            pl.BlockSpec((1, E), lambda c: (0, 0)),
        ],
        out_specs=[
            pl.BlockSpec((CT, E), lambda c: (c, 0)),
            pl.BlockSpec((CT, E), lambda c: (c, 0)),
            pl.BlockSpec((CT, E), lambda c: (c, 0)),
            pl.BlockSpec((CT, 16), lambda c: (c, 0)),
            pl.BlockSpec((CT, 16), lambda c: (c, 0)),
            pl.BlockSpec((CT, E), lambda c: (c, 0)),
            pl.BlockSpec((CT, E), lambda c: (c, 0)),
            pl.BlockSpec((CT, E), lambda c: (c, 0)),
            pl.BlockSpec((1, E), lambda c: (0, 0)),
        ],
        out_shape=[
            jax.ShapeDtypeStruct((N, E), jnp.float32),
            jax.ShapeDtypeStruct((N, E), jnp.int32),
            jax.ShapeDtypeStruct((N, E), jnp.float32),
            jax.ShapeDtypeStruct((N, 16), jnp.float32),
            jax.ShapeDtypeStruct((N, 16), jnp.float32),
            jax.ShapeDtypeStruct((N, E), jnp.float32),
            jax.ShapeDtypeStruct((N, E), jnp.float32),
            jax.ShapeDtypeStruct((N, E), jnp.float32),
            jax.ShapeDtypeStruct((1, E), jnp.float32),
        ],
    )(flat, router_W, router_b2d)


def _dst_body(oh1_ref, oh2_ref, off_ref, totals_ref,
              dst0_ref, dst1_ref, te_ref):
    tot = totals_ref[...]
    padded = jnp.ceil(tot * (1.0 / T)) * float(T)
    ru = lax.broadcasted_iota(jnp.int32, (E, E), 0)
    cu = lax.broadcasted_iota(jnp.int32, (E, E), 1)
    triu = (ru < cu).astype(jnp.float32)
    base = jnp.dot(padded, triu, preferred_element_type=jnp.float32)

    off = off_ref[...] + base
    d0 = jnp.sum(oh1_ref[...] * off, axis=1)
    d1 = jnp.sum(oh2_ref[...] * off, axis=1)
    dst0_ref[...] = d0.reshape(32, 128).astype(jnp.int32)
    dst1_ref[...] = d1.reshape(32, 128).astype(jnp.int32)

    eye = (ru == cu).astype(jnp.float32)
    base_col = jnp.sum(jnp.dot(jnp.ones((E, 1), jnp.float32), base,
                               preferred_element_type=jnp.float32) * eye,
                       axis=1, keepdims=True)
    jT = lax.broadcasted_iota(jnp.int32, (E, 64), 1).astype(jnp.float32) * float(T)
    te = jnp.sum((jT >= base_col).astype(jnp.float32), axis=0, keepdims=True) - 1.0
    total_padded = jnp.sum(padded, axis=1, keepdims=True)
    jT1 = lax.broadcasted_iota(jnp.int32, (1, 64), 1).astype(jnp.float32) * float(T)
    te = jnp.where(jT1 >= total_padded, float(E), te)
    te_ref[...] = te.astype(jnp.int32)


def _dst_call(oh1, oh2, off, totals):
    return pl.pallas_call(
        _dst_body,
        in_specs=[
            pl.BlockSpec((N, E), lambda: (0, 0)),
            pl.BlockSpec((N, E), lambda: (0, 0)),
            pl.BlockSpec((N, E), lambda: (0, 0)),
            pl.BlockSpec((1, E), lambda: (0, 0)),
        ],
        out_specs=[
            pl.BlockSpec((32, 128), lambda: (0, 0)),
            pl.BlockSpec((32, 128), lambda: (0, 0)),
            pl.BlockSpec((1, 64), lambda: (0, 0)),
        ],
        out_shape=[
            jax.ShapeDtypeStruct((32, 128), jnp.int32),
            jax.ShapeDtypeStruct((32, 128), jnp.int32),
            jax.ShapeDtypeStruct((1, 64), jnp.int32),
        ],
    )(oh1, oh2, off, totals)


def _sc_mesh():
    return plsc.VectorSubcoreMesh(core_axis_name="c", subcore_axis_name="s")


_NCORES = 2
_NSUB = 16
_NW = _NCORES * _NSUB
_TPW = N // _NW
_DCH = 32
_CCH = 16


def _dispatch_body(flat_hbm, dst0_hbm, dst1_hbm, xg_hbm,
                   rows_v, idx0_v, idx1_v, sem):
    wid = lax.axis_index("s") * _NCORES + lax.axis_index("c")
    base = wid * _TPW
    for j in range(_TPW // _DCH):
        b = base + j * _DCH
        pltpu.sync_copy(flat_hbm.at[pl.ds(b, _DCH)], rows_v)
        pltpu.sync_copy(dst0_hbm.at[pl.ds(b, _DCH)], idx0_v)
        pltpu.sync_copy(dst1_hbm.at[pl.ds(b, _DCH)], idx1_v)
        c0 = pltpu.async_copy(rows_v, xg_hbm.at[idx0_v], sem)
        c1 = pltpu.async_copy(rows_v, xg_hbm.at[idx1_v], sem)
        c0.wait()
        c1.wait()


def _dispatch_call(flat, dst0, dst1):
    f = functools.partial(
        pl.kernel,
        mesh=_sc_mesh(),
        out_type=jax.ShapeDtypeStruct((NUM_ROUTED_SLOTS, D), jnp.float32),
        scratch_types=[
            pltpu.VMEM((_DCH, D), jnp.float32),
            pltpu.VMEM((_DCH,), jnp.int32),
            pltpu.VMEM((_DCH,), jnp.int32),
            pltpu.SemaphoreType.DMA,
        ],
    )(_dispatch_body)
    return f(flat, dst0, dst1)


def _ffn_body_first(te_ref, x_ref, w1_ref, b1_ref, w2_ref, b2_ref, out_ref):
    tv = te_ref[pl.program_id(0)]

    @pl.when(tv < E)
    def _():
        x = x_ref[...]
        mid = _gelu(jnp.dot(x, w1_ref[0], preferred_element_type=jnp.float32)
                    + b1_ref[0, 0])
        out_ref[...] = (jnp.dot(mid, w2_ref[0],
                                preferred_element_type=jnp.float32)
                        + b2_ref[0])


def _ffn_body_rest(te_ref, x_ref, w1_ref, b1_ref, w2_ref, prev_ref, out_ref):
    tv = te_ref[pl.program_id(0)]

    @pl.when(tv < E)
    def _():
        x = x_ref[...]
        mid = _gelu(jnp.dot(x, w1_ref[0], preferred_element_type=jnp.float32)
                    + b1_ref[0, 0])
        out_ref[...] = prev_ref[...] + jnp.dot(
            mid, w2_ref[0], preferred_element_type=jnp.float32)


def _ffn_call(hc, te_r, xg, rW1, rb1, rW2, rb2, prev):
    first = prev is None

    def _e(te, i):
        return jnp.minimum(te[i], E - 1)

    in_specs = [
        pl.BlockSpec((T, D), lambda i, te: (i, 0)),
        pl.BlockSpec((1, D, HC), lambda i, te, _hc=hc: (_e(te, i), 0, _hc)),
        pl.BlockSpec((1, 1, 1, HC),
                     lambda i, te, _hc=hc: (_e(te, i), _hc, 0, 0)),
        pl.BlockSpec((1, HC, D), lambda i, te, _hc=hc: (_e(te, i), _hc, 0)),
    ]
    args = [xg, rW1, rb1.reshape(E, NHC, 1, HC), rW2]
    if first:
        body = _ffn_body_first
        in_specs.append(pl.BlockSpec((1, 1, D), lambda i, te: (_e(te, i), 0, 0)))
        args.append(rb2.reshape(E, 1, D))
    else:
        body = _ffn_body_rest
        in_specs.append(pl.BlockSpec((T, D), lambda i, te: (i, 0)))
        args.append(prev)
    return pl.pallas_call(
        body,
        grid_spec=pltpu.PrefetchScalarGridSpec(
            num_scalar_prefetch=1,
            grid=(NT_ROUTED,),
            in_specs=in_specs,
            out_specs=pl.BlockSpec((T, D), lambda i, te: (i, 0)),
        ),
        out_shape=jax.ShapeDtypeStruct((NUM_ROUTED_SLOTS, D), jnp.float32),
        compiler_params=pltpu.CompilerParams(
            vmem_limit_bytes=128 * 1024 * 1024),
    )(te_r, *args)


def _sffn_body_first(x_ref, w1_ref, b1_ref, w2_ref, b2_ref, out_ref):
    x = x_ref[...]
    mid = _gelu(jnp.dot(x, w1_ref[...], preferred_element_type=jnp.float32)
                + b1_ref[...])
    out_ref[...] = (jnp.dot(mid, w2_ref[...],
                            preferred_element_type=jnp.float32) + b2_ref[...])


def _sffn_body_rest(x_ref, w1_ref, b1_ref, w2_ref, prev_ref, out_ref):
    x = x_ref[...]
    mid = _gelu(jnp.dot(x, w1_ref[...], preferred_element_type=jnp.float32)
                + b1_ref[...])
    out_ref[...] = prev_ref[...] + jnp.dot(
        mid, w2_ref[...], preferred_element_type=jnp.float32)


def _sffn_call(hc, flat, sW1, sb1, sW2, sb2, prev):
    first = prev is None
    in_specs = [
        pl.BlockSpec((T, D), lambda i: (i, 0)),
        pl.BlockSpec((D, HC), lambda i, _hc=hc: (0, _hc)),
        pl.BlockSpec((1, HC), lambda i, _hc=hc: (0, _hc)),
        pl.BlockSpec((HC, D), lambda i, _hc=hc: (_hc, 0)),
    ]
    args = [flat, sW1[0], sb1, sW2[0]]
    if first:
        body = _sffn_body_first
        in_specs.append(pl.BlockSpec((1, D), lambda i: (0, 0)))
        args.append(sb2)
    else:
        body = _sffn_body_rest
        in_specs.append(pl.BlockSpec((T, D), lambda i: (i, 0)))
        args.append(prev)
    return pl.pallas_call(
        body,
        grid=(NT_SHARED,),
        in_specs=in_specs,
        out_specs=pl.BlockSpec((T, D), lambda i: (i, 0)),
        out_shape=jax.ShapeDtypeStruct((N, D), jnp.float32),
        compiler_params=pltpu.CompilerParams(
            vmem_limit_bytes=128 * 1024 * 1024),
    )(*args)


def _combine_body(y_hbm, ys_hbm, dst0_hbm, dst1_hbm, p0b_hbm, p1b_hbm,
                  out_hbm, acc_v, buf0_v, buf1_v, idx0_v, idx1_v, w0_v, w1_v,
                  sem):
    wid = lax.axis_index("s") * _NCORES + lax.axis_index("c")
    base = wid * _TPW
    for j in range(_TPW // _CCH):
        b = base + j * _CCH
        pltpu.sync_copy(dst0_hbm.at[pl.ds(b, _CCH)], idx0_v)
        pltpu.sync_copy(dst1_hbm.at[pl.ds(b, _CCH)], idx1_v)
        pltpu.sync_copy(p0b_hbm.at[pl.ds(b, _CCH)], w0_v)
        pltpu.sync_copy(p1b_hbm.at[pl.ds(b, _CCH)], w1_v)
        pltpu.sync_copy(ys_hbm.at[pl.ds(b, _CCH)], acc_v)
        g0 = pltpu.async_copy(y_hbm.at[idx0_v], buf0_v, sem)
        g1 = pltpu.async_copy(y_hbm.at[idx1_v], buf1_v, sem)
        g0.wait()
        g1.wait()

        w0s = [w0_v[r] for r in range(_CCH)]
        w1s = [w1_v[r] for r in range(_CCH)]

        def col_body(cb, _):
            sl = pl.ds(cb * 16, 16)
            for r in range(_CCH):
                acc_v.at[r][sl] = (acc_v.at[r][sl]
                                   + w0s[r] * buf0_v.at[r][sl]
                                   + w1s[r] * buf1_v.at[r][sl])
            return _

        lax.fori_loop(0, D // 16, col_body, 0)
        pltpu.sync_copy(acc_v, out_hbm.at[pl.ds(b, _CCH)])


def _combine_call(y, ys, dst0, dst1, p0b, p1b):
    f = functools.partial(
        pl.kernel,
        mesh=_sc_mesh(),
        out_type=jax.ShapeDtypeStruct((N, D), jnp.float32),
        scratch_types=[
            pltpu.VMEM((_CCH, D), jnp.float32),
            pltpu.VMEM((_CCH, D), jnp.float32),
            pltpu.VMEM((_CCH, D), jnp.float32),
            pltpu.VMEM((_CCH,), jnp.int32),
            pltpu.VMEM((_CCH,), jnp.int32),
            pltpu.VMEM((_CCH, 16), jnp.float32),
            pltpu.VMEM((_CCH, 16), jnp.float32),
            pltpu.SemaphoreType.DMA,
        ],
    )(_combine_body)
    return f(y, ys, dst0, dst1, p0b, p1b)


def kernel(hidden, router_W, router_b, shared_W1, shared_b1, shared_W2,
           shared_b2, routed_W1, routed_b1, routed_W2, routed_b2):
    flat = hidden.reshape(N, D)

    (logits, idx8, probs8, p0b, p1b,
     oh1, oh2, off, totals) = _router_call(flat, router_W,
                                           router_b.reshape(1, E))
    dst0m, dst1m, te_pad = _dst_call(oh1, oh2, off, totals)
    dst0 = dst0m.reshape(N)
    dst1 = dst1m.reshape(N)
    te_r = te_pad.reshape(64)[:NT_ROUTED]

    xg = _dispatch_call(flat, dst0, dst1)

    y = None
    ys = None
    for hc in range(NHC):
        y = _ffn_call(hc, te_r, xg, routed_W1, routed_b1, routed_W2,
                      routed_b2, y)
        ys = _sffn_call(hc, flat, shared_W1, shared_b1, shared_W2,
                        shared_b2, ys)

    out_flat = _combine_call(y, ys, dst0, dst1, p0b, p1b)

    output = out_flat.reshape(B, S, D)
    aux_logits = logits.reshape(B, S, E)
    aux_idx = idx8[:, :2].reshape(B, S, 2)
    aux_probs = probs8[:, :2].reshape(B, S, 2)
    return output, aux_logits, aux_idx, aux_probs
